# initial kernel scaffold (unmeasured)
import jax
import jax.numpy as jnp
from jax import lax
from jax.experimental import pallas as pl
from jax.experimental.pallas import tpu as pltpu


def kernel(
    x,
):
    def body(*refs):
        pass

    out_shape = jax.ShapeDtypeStruct(..., jnp.float32)
    return pl.pallas_call(body, out_shape=out_shape)(...)


# baseline (device time: 31873 ns/iter reference)
import jax
import jax.numpy as jnp
from jax import lax
from jax.experimental import pallas as pl
from jax.experimental.pallas import tpu as pltpu


def kernel(x):
    M, N = x.shape
    dt = x.dtype

    def body(
        x_ref,
        out_ref,
        row_send,
        col_send,
        row_recv,
        col_recv,
        send_sems,
        recv_sems,
        ack_sem,
    ):
        sx = lax.axis_index("x")
        sy = lax.axis_index("y")

        @pl.when(sx == 0)
        def _():
            row_send[...] = x_ref[M - 1 : M, :]

        @pl.when(sx == 1)
        def _():
            row_send[...] = x_ref[0:1, :]

        @pl.when(sy == 0)
        def _():
            col_send[...] = x_ref[:, N - 1 : N]

        @pl.when(sy == 1)
        def _():
            col_send[...] = x_ref[:, 0:1]

        rdma_x = pltpu.make_async_remote_copy(
            src_ref=row_send,
            dst_ref=row_recv,
            send_sem=send_sems.at[0],
            recv_sem=recv_sems.at[0],
            device_id=(1 - sx, sy),
            device_id_type=pl.DeviceIdType.MESH,
        )
        rdma_y = pltpu.make_async_remote_copy(
            src_ref=col_send,
            dst_ref=col_recv,
            send_sem=send_sems.at[1],
            recv_sem=recv_sems.at[1],
            device_id=(sx, 1 - sy),
            device_id_type=pl.DeviceIdType.MESH,
        )
        rdma_x.start()
        rdma_y.start()

        out_ref[...] = 0.5 * x_ref[...]
        out_ref[1:M, :] = out_ref[1:M, :] + 0.125 * x_ref[0 : M - 1, :]
        out_ref[0 : M - 1, :] = out_ref[0 : M - 1, :] + 0.125 * x_ref[1:M, :]
        out_ref[:, 1:N] = out_ref[:, 1:N] + 0.125 * x_ref[:, 0 : N - 1]
        out_ref[:, 0 : N - 1] = out_ref[:, 0 : N - 1] + 0.125 * x_ref[:, 1:N]

        rdma_x.wait()

        @pl.when(sx == 0)
        def _():
            out_ref[M - 1 : M, :] = out_ref[M - 1 : M, :] + 0.125 * row_recv[...]

        @pl.when(sx == 1)
        def _():
            out_ref[0:1, :] = out_ref[0:1, :] + 0.125 * row_recv[...]

        rdma_y.wait()

        @pl.when(sy == 0)
        def _():
            out_ref[:, N - 1 : N] = out_ref[:, N - 1 : N] + 0.125 * col_recv[...]

        @pl.when(sy == 1)
        def _():
            out_ref[:, 0:1] = out_ref[:, 0:1] + 0.125 * col_recv[...]

        @pl.when(sx == 0)
        def _():
            out_ref[0:1, :] = x_ref[0:1, :]

        @pl.when(sx == 1)
        def _():
            out_ref[M - 1 : M, :] = x_ref[M - 1 : M, :]

        @pl.when(sy == 0)
        def _():
            out_ref[:, 0:1] = x_ref[:, 0:1]

        @pl.when(sy == 1)
        def _():
            out_ref[:, N - 1 : N] = x_ref[:, N - 1 : N]

        pl.semaphore_signal(
            ack_sem,
            inc=1,
            device_id=(1 - sx, sy),
            device_id_type=pl.DeviceIdType.MESH,
        )
        pl.semaphore_signal(
            ack_sem,
            inc=1,
            device_id=(sx, 1 - sy),
            device_id_type=pl.DeviceIdType.MESH,
        )
        pl.semaphore_wait(ack_sem, 2)

    return pl.pallas_call(
        body,
        out_shape=jax.ShapeDtypeStruct((M, N), dt),
        in_specs=[pl.BlockSpec(memory_space=pltpu.VMEM)],
        out_specs=pl.BlockSpec(memory_space=pltpu.VMEM),
        scratch_shapes=[
            pltpu.VMEM((1, N), dt),
            pltpu.VMEM((M, 1), dt),
            pltpu.VMEM((1, N), dt),
            pltpu.VMEM((M, 1), dt),
            pltpu.SemaphoreType.DMA((2,)),
            pltpu.SemaphoreType.DMA((2,)),
            pltpu.SemaphoreType.REGULAR,
        ],
        compiler_params=pltpu.CompilerParams(has_side_effects=True),
    )(x)
